# baseline (speedup 1.0000x reference)
import functools

import jax
import jax.numpy as jnp
from jax import lax
from jax.experimental import pallas as pl
from jax.experimental.pallas import tpu as pltpu
from jax.experimental.pallas import tpu_sc as plsc

D_MODEL = 64
SCALE = 8.0

NC = 2
NS = 16
NW = NC * NS
LANES = 16

CHUNK = 128


def _make_kernel(n_idx):
    assert n_idx % (NW * CHUNK) == 0
    b_per_w = n_idx // NW
    steps = b_per_w // CHUNK

    mesh = plsc.VectorSubcoreMesh(core_axis_name="c", subcore_axis_name="s")

    @functools.partial(
        pl.kernel,
        out_type=jax.ShapeDtypeStruct((n_idx, D_MODEL), jnp.float32),
        mesh=mesh,
        compiler_params=pltpu.CompilerParams(use_tc_tiling_on_sc=False),
        scratch_types=[
            pltpu.VMEM((CHUNK,), jnp.int32),
            pltpu.VMEM((CHUNK, D_MODEL), jnp.float32),
            pltpu.SemaphoreType.DMA,
        ],
    )
    def emb_kernel(x_hbm, table_hbm, out_hbm, idx_v, rows_v, gsem):
        wid = lax.axis_index("s") * NC + lax.axis_index("c")
        base = wid * b_per_w

        def scale_row(r, carry):
            for c in range(D_MODEL // LANES):
                sl = pl.ds(c * LANES, LANES)
                rows_v[r, sl] = rows_v[r, sl] * SCALE
            return carry

        def step(s, carry):
            off = base + s * CHUNK
            pltpu.sync_copy(x_hbm.at[pl.ds(off, CHUNK)], idx_v)
            pltpu.async_copy(table_hbm.at[idx_v], rows_v, gsem).wait()
            lax.fori_loop(0, CHUNK, scale_row, 0)
            pltpu.sync_copy(rows_v, out_hbm.at[pl.ds(off, CHUNK)])
            return carry

        lax.fori_loop(0, steps, step, 0)

    return emb_kernel


def kernel(x, table):
    orig_shape = x.shape
    x_flat = x.reshape(-1).astype(jnp.int32)
    out = _make_kernel(x_flat.shape[0])(x_flat, table)
    return out.reshape(*orig_shape, D_MODEL)
